# baseline (speedup 1.0000x reference)
import functools

import jax
import jax.numpy as jnp
from jax import lax
from jax.experimental import pallas as pl
from jax.experimental.pallas import tpu as pltpu
from jax.experimental.pallas import tpu_sc as plsc

NC = 2
NS = 16
NW = NC * NS
L = 16

_SLOPE = 0.2
_PKSH = 14


def _leaky(x):
    return jnp.maximum(x, _SLOPE * x)


def _nodes_body(x_ref, ww_ref, bw_ref, wa_ref, ei_ref, htp_ref, s1_ref,
                s2_ref, mx_ref, pk_ref):
    h = _leaky(jnp.dot(x_ref[...], ww_ref[...],
                       preferred_element_type=jnp.float32) + bw_ref[...])
    ht = h.T
    s = jnp.dot(wa_ref[...], ht, preferred_element_type=jnp.float32)
    s1_ref[...] = s[0]
    s2_ref[...] = s[1]
    mx_ref[...] = jnp.max(s, axis=1)
    dh, n = ht.shape
    hb3 = ht.astype(jnp.bfloat16).reshape(dh // 2, 2, n)
    lo = lax.bitcast_convert_type(hb3[:, 0, :], jnp.uint16).astype(jnp.uint32)
    hi = lax.bitcast_convert_type(hb3[:, 1, :], jnp.uint16).astype(jnp.uint32)
    pk = lo | (hi << 16)
    htp_ref[...] = lax.bitcast_convert_type(pk, jnp.float32)
    ei = ei_ref[...]
    pk_ref[...] = (ei[0] << _PKSH) | ei[1]


def _edges_body(ef_ref, wes_ref, bs_ref, wa3_ref, ae_ref, mx_ref):
    y = lax.dot_general(wes_ref[...].astype(jnp.bfloat16),
                        ef_ref[...].astype(jnp.bfloat16),
                        (((0,), (1,)), ((), ())),
                        preferred_element_type=jnp.float32) + bs_ref[...]
    sel = jnp.where(wa3_ref[...] >= 0.0,
                    jnp.maximum(y, _SLOPE * y),
                    jnp.minimum(y, _SLOPE * y))
    ae = jnp.sum(sel, axis=0)
    be = ae.shape[0]
    ae_ref[...] = ae.reshape(1, 1, be)
    mx_ref[...] = jnp.max(ae).reshape(1, 1, 1)


def _rden_body(dp_ref, rden_ref):
    rden_ref[...] = 1.0 / jnp.sum(dp_ref[...], axis=0)


def _fin_body(ot_ref, out_ref):
    out_ref[...] = _leaky(ot_ref[...].T)


def _sc_attn_body(s1_hbm, s2_hbm, ae_hbm, shift_hbm, pk_hbm,
                  ex_out, denp_out,
                  s1_v, s2_v, den_v, pk_v, ae_v, ex_v, sh_v,
                  *, n_nodes, epw):
    wid = lax.axis_index("s") * NC + lax.axis_index("c")
    base = wid * epw
    pltpu.sync_copy(s1_hbm, s1_v)
    pltpu.sync_copy(s2_hbm, s2_v)
    pltpu.sync_copy(pk_hbm.at[pl.ds(base, epw)], pk_v)
    pltpu.sync_copy(ae_hbm.at[pl.ds(base, epw)], ae_v)
    pltpu.sync_copy(shift_hbm, sh_v)
    shift = sh_v[...]

    zeros = jnp.zeros((L,), jnp.float32)
    nmask = jnp.full((L,), (1 << _PKSH) - 1, jnp.int32)
    shv = jnp.full((L,), _PKSH, jnp.int32)

    @plsc.parallel_loop(0, n_nodes // L, unroll=8)
    def _(i):
        den_v[pl.ds(i * L, L)] = zeros

    @plsc.parallel_loop(0, epw // L, unroll=4)
    def _(j):
        sl = pl.ds(j * L, L)
        pk = pk_v[sl]
        t = lax.shift_right_logical(pk, shv)
        n = pk & nmask
        v1 = plsc.load_gather(s1_v, [t])
        v2 = plsc.load_gather(s2_v, [n])
        ex = jnp.exp(v1 + v2 + ae_v[sl] + shift)
        ex_v[sl] = ex
        plsc.addupdate_scatter(den_v, [t], ex)

    pltpu.sync_copy(ex_v, ex_out.at[pl.ds(base, epw)])
    pltpu.sync_copy(den_v, denp_out.at[pl.ds(wid * n_nodes, n_nodes)])


def _sc_agg_body(htp_hbm, rden_hbm, ex_hbm, pk_hbm,
                 outp,
                 ht2_v, ht_v, out_v, rden_v,
                 pk0, ex0, pk1, ex1, sem0, sem1,
                 *, n_nodes, n_edges, cpw, chunk):
    wid = lax.axis_index("s") * NC + lax.axis_index("c")
    cpw2 = cpw // 2
    pltpu.sync_copy(htp_hbm.at[pl.ds(wid * cpw2, cpw2)], ht2_v)
    pltpu.sync_copy(rden_hbm, rden_v)

    @plsc.parallel_loop(0, n_nodes // L, unroll=8)
    def _(i):
        for c2 in range(cpw2):
            ht_v[pl.ds(c2 * n_nodes + i * L, L)] = ht2_v[c2, pl.ds(i * L, L)]

    zeros = jnp.zeros((L,), jnp.float32)
    stripe = n_nodes * cpw
    nchunks = n_edges // chunk
    nmask = jnp.full((L,), (1 << _PKSH) - 1, jnp.int32)
    shv = jnp.full((L,), _PKSH, jnp.int32)
    sh16 = jnp.full((L,), 16, jnp.int32)
    himask = jnp.full((L,), -65536, jnp.int32)

    @plsc.parallel_loop(0, stripe // L, unroll=8)
    def _(i):
        out_v[pl.ds(i * L, L)] = zeros

    wrd_base = [jnp.full((L,), c2 * n_nodes, jnp.int32) for c2 in range(cpw2)]
    col_base = [jnp.full((L,), c * n_nodes, jnp.int32) for c in range(cpw)]

    def _start(kc, pb, xb, sem):
        cb = kc * chunk
        pltpu.async_copy(pk_hbm.at[pl.ds(cb, chunk)], pb, sem)
        pltpu.async_copy(ex_hbm.at[pl.ds(cb, chunk)], xb, sem)

    def _wait(kc, pb, xb, sem):
        cb = kc * chunk
        pltpu.make_async_copy(pk_hbm.at[pl.ds(cb, chunk)], pb, sem).wait()
        pltpu.make_async_copy(ex_hbm.at[pl.ds(cb, chunk)], xb, sem).wait()

    def _consume(pb, xb):
        @plsc.parallel_loop(0, chunk // L, unroll=8)
        def _(j):
            sl = pl.ds(j * L, L)
            pk = pb[sl]
            t = lax.shift_right_logical(pk, shv)
            n = pk & nmask
            att = xb[sl] * plsc.load_gather(rden_v, [t])
            for c2 in range(cpw2):
                w = plsc.bitcast(plsc.load_gather(ht_v, [n + wrd_base[c2]]),
                                 jnp.int32)
                hlo = plsc.bitcast(lax.shift_left(w, sh16), jnp.float32)
                hhi = plsc.bitcast(w & himask, jnp.float32)
                plsc.addupdate_scatter(out_v, [t + col_base[2 * c2]],
                                       att * hlo)
                plsc.addupdate_scatter(out_v, [t + col_base[2 * c2 + 1]],
                                       att * hhi)

    _start(0, pk0, ex0, sem0)

    def chunk_body(k2, c):
        c0 = 2 * k2
        _start(c0 + 1, pk1, ex1, sem1)
        _wait(c0, pk0, ex0, sem0)
        _consume(pk0, ex0)

        @pl.when(c0 + 2 < nchunks)
        def _():
            _start(c0 + 2, pk0, ex0, sem0)

        _wait(c0 + 1, pk1, ex1, sem1)
        _consume(pk1, ex1)
        return c

    lax.fori_loop(0, nchunks // 2, chunk_body, 0)

    pltpu.sync_copy(out_v, outp.at[pl.ds(wid * stripe, stripe)])


def kernel(node_features, edge_features, W_w, b_w, W_e, b_e, W_a, b_a,
           edge_index):
    N, Df = node_features.shape
    E, De = edge_features.shape
    Dh = W_w.shape[1]
    f32 = jnp.float32

    assert N % L == 0 and E % NW == 0 and Dh % NW == 0
    assert N <= (1 << _PKSH)
    epw = E // NW
    cpw = Dh // NW
    chunk = 8000
    assert E % (2 * chunk) == 0 and chunk % L == 0 and epw % L == 0

    wa = W_a[:, 0]
    wa12 = wa[:2 * Dh].reshape(2, Dh)

    htp, s1, s2, mx12, pk = pl.pallas_call(
        _nodes_body,
        out_shape=[
            jax.ShapeDtypeStruct((Dh // 2, N), f32),
            jax.ShapeDtypeStruct((N,), f32),
            jax.ShapeDtypeStruct((N,), f32),
            jax.ShapeDtypeStruct((2,), f32),
            jax.ShapeDtypeStruct((E,), jnp.int32),
        ],
    )(node_features, W_w, b_w.reshape(1, Dh), wa12, edge_index)

    BE = 6400
    nbe = E // BE
    wes = W_e * wa[2 * Dh:][None, :]
    bs_col = (b_e * wa[2 * Dh:]).reshape(Dh, 1)
    wa3_col = wa[2 * Dh:].reshape(Dh, 1)
    ae3, mxb = pl.pallas_call(
        _edges_body,
        grid=(nbe,),
        in_specs=[
            pl.BlockSpec((BE, De), lambda i: (i, 0)),
            pl.BlockSpec((De, Dh), lambda i: (0, 0)),
            pl.BlockSpec((Dh, 1), lambda i: (0, 0)),
            pl.BlockSpec((Dh, 1), lambda i: (0, 0)),
        ],
        out_specs=[
            pl.BlockSpec((1, 1, BE), lambda i: (i, 0, 0)),
            pl.BlockSpec((1, 1, 1), lambda i: (i, 0, 0)),
        ],
        out_shape=[
            jax.ShapeDtypeStruct((nbe, 1, BE), f32),
            jax.ShapeDtypeStruct((nbe, 1, 1), f32),
        ],
    )(edge_features, wes, bs_col, wa3_col)
    ae = ae3.reshape(E)

    bound = mx12[0] + mx12[1] + jnp.max(mxb)
    shift = jnp.full((L,), 0.0, f32) - bound

    mesh = plsc.VectorSubcoreMesh(core_axis_name="c", subcore_axis_name="s")
    sc_params = pltpu.CompilerParams(needs_layout_passes=False)
    sc_attn = pl.kernel(
        functools.partial(_sc_attn_body, n_nodes=N, epw=epw),
        mesh=mesh,
        compiler_params=sc_params,
        out_type=(
            jax.ShapeDtypeStruct((E,), f32),
            jax.ShapeDtypeStruct((NW * N,), f32),
        ),
        scratch_types=[
            pltpu.VMEM((N,), f32),
            pltpu.VMEM((N,), f32),
            pltpu.VMEM((N,), f32),
            pltpu.VMEM((epw,), jnp.int32),
            pltpu.VMEM((epw,), f32),
            pltpu.VMEM((epw,), f32),
            pltpu.VMEM((L,), f32),
        ],
    )
    ex, denp = sc_attn(s1, s2, ae, shift, pk)

    rden = pl.pallas_call(
        _rden_body,
        out_shape=jax.ShapeDtypeStruct((N,), f32),
    )(denp.reshape(NW, N))

    sc_agg = pl.kernel(
        functools.partial(_sc_agg_body, n_nodes=N, n_edges=E, cpw=cpw,
                          chunk=chunk),
        mesh=mesh,
        compiler_params=sc_params,
        out_type=jax.ShapeDtypeStruct((NW * N * cpw,), f32),
        scratch_types=[
            pltpu.VMEM((cpw // 2, N), f32),
            pltpu.VMEM((N * cpw // 2,), f32),
            pltpu.VMEM((N * cpw,), f32),
            pltpu.VMEM((N,), f32),
            pltpu.VMEM((chunk,), jnp.int32),
            pltpu.VMEM((chunk,), f32),
            pltpu.VMEM((chunk,), jnp.int32),
            pltpu.VMEM((chunk,), f32),
            pltpu.SemaphoreType.DMA,
            pltpu.SemaphoreType.DMA,
        ],
    )
    outp = sc_agg(htp, rden, ex, pk)

    out = pl.pallas_call(
        _fin_body,
        out_shape=jax.ShapeDtypeStruct((N, Dh), f32),
    )(outp.reshape(Dh, N))
    return out


# speedup vs baseline: 1.0464x; 1.0464x over previous
import functools

import jax
import jax.numpy as jnp
from jax import lax
from jax.experimental import pallas as pl
from jax.experimental.pallas import tpu as pltpu
from jax.experimental.pallas import tpu_sc as plsc

NC = 2
NS = 16
NW = NC * NS
L = 16

_SLOPE = 0.2
_PKSH = 14


def _leaky(x):
    return jnp.maximum(x, _SLOPE * x)


def _nodes_body(x_ref, ww_ref, bw_ref, wa_ref, ei_ref, htp_ref, s1_ref,
                s2_ref, mx_ref, pk_ref):
    h = _leaky(jnp.dot(x_ref[...], ww_ref[...],
                       preferred_element_type=jnp.float32) + bw_ref[...])
    ht = h.T
    s = jnp.dot(wa_ref[...], ht, preferred_element_type=jnp.float32)
    s1_ref[...] = s[0]
    s2_ref[...] = s[1]
    mx_ref[...] = jnp.max(s, axis=1)
    dh, n = ht.shape
    hb3 = ht.astype(jnp.bfloat16).reshape(dh // 2, 2, n)
    lo = lax.bitcast_convert_type(hb3[:, 0, :], jnp.uint16).astype(jnp.uint32)
    hi = lax.bitcast_convert_type(hb3[:, 1, :], jnp.uint16).astype(jnp.uint32)
    pk = lo | (hi << 16)
    htp_ref[...] = lax.bitcast_convert_type(pk, jnp.float32)
    ei = ei_ref[...]
    pk_ref[...] = (ei[0] << _PKSH) | ei[1]


def _edges_body(ef_ref, wes_ref, bs_ref, wa3_ref, ae_ref, mx_ref):
    y = lax.dot_general(wes_ref[...].astype(jnp.bfloat16),
                        ef_ref[...].astype(jnp.bfloat16),
                        (((0,), (1,)), ((), ())),
                        preferred_element_type=jnp.float32) + bs_ref[...]
    sel = jnp.where(wa3_ref[...] >= 0.0,
                    jnp.maximum(y, _SLOPE * y),
                    jnp.minimum(y, _SLOPE * y))
    ae = jnp.sum(sel, axis=0)
    be = ae.shape[0]
    ae_ref[...] = ae.reshape(1, 1, be)
    mx_ref[...] = jnp.max(ae).reshape(1, 1, 1)


def _rden_body(dp_ref, rden_ref):
    rden_ref[...] = 1.0 / jnp.sum(dp_ref[...], axis=0)


def _fin_body(ot_ref, rden_ref, out_ref):
    scaled = ot_ref[...] * rden_ref[...].reshape(1, -1)
    out_ref[...] = _leaky(scaled.T)


def _sc_attn_body(s1_hbm, s2_hbm, ae_hbm, shift_hbm, pk_hbm,
                  ex_out, denp_out,
                  s1_v, s2_v, den_v, pk_v, ae_v, ex_v, sh_v,
                  *, n_nodes, epw):
    wid = lax.axis_index("s") * NC + lax.axis_index("c")
    base = wid * epw
    pltpu.sync_copy(s1_hbm, s1_v)
    pltpu.sync_copy(s2_hbm, s2_v)
    pltpu.sync_copy(pk_hbm.at[pl.ds(base, epw)], pk_v)
    pltpu.sync_copy(ae_hbm.at[pl.ds(base, epw)], ae_v)
    pltpu.sync_copy(shift_hbm, sh_v)
    shift = sh_v[...]

    zeros = jnp.zeros((L,), jnp.float32)
    nmask = jnp.full((L,), (1 << _PKSH) - 1, jnp.int32)
    shv = jnp.full((L,), _PKSH, jnp.int32)

    @plsc.parallel_loop(0, n_nodes // L, unroll=8)
    def _(i):
        den_v[pl.ds(i * L, L)] = zeros

    @plsc.parallel_loop(0, epw // L, unroll=4)
    def _(j):
        sl = pl.ds(j * L, L)
        pk = pk_v[sl]
        t = lax.shift_right_logical(pk, shv)
        n = pk & nmask
        v1 = plsc.load_gather(s1_v, [t])
        v2 = plsc.load_gather(s2_v, [n])
        ex = jnp.exp(v1 + v2 + ae_v[sl] + shift)
        ex_v[sl] = ex
        plsc.addupdate_scatter(den_v, [t], ex)

    pltpu.sync_copy(ex_v, ex_out.at[pl.ds(base, epw)])
    pltpu.sync_copy(den_v, denp_out.at[pl.ds(wid * n_nodes, n_nodes)])


def _sc_agg_body(htp_hbm, ex_hbm, pk_hbm,
                 outp,
                 ht2_v, ht_v, out_v,
                 pk0, ex0, pk1, ex1, sem0, sem1,
                 *, n_nodes, n_edges, cpw, chunk):
    wid = lax.axis_index("s") * NC + lax.axis_index("c")
    cpw2 = cpw // 2
    pltpu.sync_copy(htp_hbm.at[pl.ds(wid * cpw2, cpw2)], ht2_v)

    @plsc.parallel_loop(0, n_nodes // L, unroll=8)
    def _(i):
        for c2 in range(cpw2):
            ht_v[pl.ds(c2 * n_nodes + i * L, L)] = ht2_v[c2, pl.ds(i * L, L)]

    zeros = jnp.zeros((L,), jnp.float32)
    stripe = n_nodes * cpw
    nchunks = n_edges // chunk
    nmask = jnp.full((L,), (1 << _PKSH) - 1, jnp.int32)
    shv = jnp.full((L,), _PKSH, jnp.int32)
    sh16 = jnp.full((L,), 16, jnp.int32)
    himask = jnp.full((L,), -65536, jnp.int32)

    @plsc.parallel_loop(0, stripe // L, unroll=8)
    def _(i):
        out_v[pl.ds(i * L, L)] = zeros

    wrd_base = [jnp.full((L,), c2 * n_nodes, jnp.int32) for c2 in range(cpw2)]
    col_base = [jnp.full((L,), c * n_nodes, jnp.int32) for c in range(cpw)]

    def _start(kc, pb, xb, sem):
        cb = kc * chunk
        pltpu.async_copy(pk_hbm.at[pl.ds(cb, chunk)], pb, sem)
        pltpu.async_copy(ex_hbm.at[pl.ds(cb, chunk)], xb, sem)

    def _wait(kc, pb, xb, sem):
        cb = kc * chunk
        pltpu.make_async_copy(pk_hbm.at[pl.ds(cb, chunk)], pb, sem).wait()
        pltpu.make_async_copy(ex_hbm.at[pl.ds(cb, chunk)], xb, sem).wait()

    def _consume(pb, xb):
        @plsc.parallel_loop(0, chunk // L, unroll=8)
        def _(j):
            sl = pl.ds(j * L, L)
            pk = pb[sl]
            t = lax.shift_right_logical(pk, shv)
            n = pk & nmask
            att = xb[sl]
            for c2 in range(cpw2):
                w = plsc.bitcast(plsc.load_gather(ht_v, [n + wrd_base[c2]]),
                                 jnp.int32)
                hlo = plsc.bitcast(lax.shift_left(w, sh16), jnp.float32)
                hhi = plsc.bitcast(w & himask, jnp.float32)
                plsc.addupdate_scatter(out_v, [t + col_base[2 * c2]],
                                       att * hlo)
                plsc.addupdate_scatter(out_v, [t + col_base[2 * c2 + 1]],
                                       att * hhi)

    _start(0, pk0, ex0, sem0)

    def chunk_body(k2, c):
        c0 = 2 * k2
        _start(c0 + 1, pk1, ex1, sem1)
        _wait(c0, pk0, ex0, sem0)
        _consume(pk0, ex0)

        @pl.when(c0 + 2 < nchunks)
        def _():
            _start(c0 + 2, pk0, ex0, sem0)

        _wait(c0 + 1, pk1, ex1, sem1)
        _consume(pk1, ex1)
        return c

    lax.fori_loop(0, nchunks // 2, chunk_body, 0)

    pltpu.sync_copy(out_v, outp.at[pl.ds(wid * stripe, stripe)])


def kernel(node_features, edge_features, W_w, b_w, W_e, b_e, W_a, b_a,
           edge_index):
    N, Df = node_features.shape
    E, De = edge_features.shape
    Dh = W_w.shape[1]
    f32 = jnp.float32

    assert N % L == 0 and E % NW == 0 and Dh % NW == 0
    assert N <= (1 << _PKSH)
    epw = E // NW
    cpw = Dh // NW
    chunk = 8000
    assert E % (2 * chunk) == 0 and chunk % L == 0 and epw % L == 0

    wa = W_a[:, 0]
    wa12 = wa[:2 * Dh].reshape(2, Dh)

    htp, s1, s2, mx12, pk = pl.pallas_call(
        _nodes_body,
        out_shape=[
            jax.ShapeDtypeStruct((Dh // 2, N), f32),
            jax.ShapeDtypeStruct((N,), f32),
            jax.ShapeDtypeStruct((N,), f32),
            jax.ShapeDtypeStruct((2,), f32),
            jax.ShapeDtypeStruct((E,), jnp.int32),
        ],
    )(node_features, W_w, b_w.reshape(1, Dh), wa12, edge_index)

    BE = 6400
    nbe = E // BE
    wes = W_e * wa[2 * Dh:][None, :]
    bs_col = (b_e * wa[2 * Dh:]).reshape(Dh, 1)
    wa3_col = wa[2 * Dh:].reshape(Dh, 1)
    ae3, mxb = pl.pallas_call(
        _edges_body,
        grid=(nbe,),
        in_specs=[
            pl.BlockSpec((BE, De), lambda i: (i, 0)),
            pl.BlockSpec((De, Dh), lambda i: (0, 0)),
            pl.BlockSpec((Dh, 1), lambda i: (0, 0)),
            pl.BlockSpec((Dh, 1), lambda i: (0, 0)),
        ],
        out_specs=[
            pl.BlockSpec((1, 1, BE), lambda i: (i, 0, 0)),
            pl.BlockSpec((1, 1, 1), lambda i: (i, 0, 0)),
        ],
        out_shape=[
            jax.ShapeDtypeStruct((nbe, 1, BE), f32),
            jax.ShapeDtypeStruct((nbe, 1, 1), f32),
        ],
    )(edge_features, wes, bs_col, wa3_col)
    ae = ae3.reshape(E)

    bound = mx12[0] + mx12[1] + jnp.max(mxb)
    shift = jnp.full((L,), 0.0, f32) - bound

    mesh = plsc.VectorSubcoreMesh(core_axis_name="c", subcore_axis_name="s")
    sc_params = pltpu.CompilerParams(needs_layout_passes=False)
    sc_attn = pl.kernel(
        functools.partial(_sc_attn_body, n_nodes=N, epw=epw),
        mesh=mesh,
        compiler_params=sc_params,
        out_type=(
            jax.ShapeDtypeStruct((E,), f32),
            jax.ShapeDtypeStruct((NW * N,), f32),
        ),
        scratch_types=[
            pltpu.VMEM((N,), f32),
            pltpu.VMEM((N,), f32),
            pltpu.VMEM((N,), f32),
            pltpu.VMEM((epw,), jnp.int32),
            pltpu.VMEM((epw,), f32),
            pltpu.VMEM((epw,), f32),
            pltpu.VMEM((L,), f32),
        ],
    )
    ex, denp = sc_attn(s1, s2, ae, shift, pk)

    rden = pl.pallas_call(
        _rden_body,
        out_shape=jax.ShapeDtypeStruct((N,), f32),
    )(denp.reshape(NW, N))

    sc_agg = pl.kernel(
        functools.partial(_sc_agg_body, n_nodes=N, n_edges=E, cpw=cpw,
                          chunk=chunk),
        mesh=mesh,
        compiler_params=sc_params,
        out_type=jax.ShapeDtypeStruct((NW * N * cpw,), f32),
        scratch_types=[
            pltpu.VMEM((cpw // 2, N), f32),
            pltpu.VMEM((N * cpw // 2,), f32),
            pltpu.VMEM((N * cpw,), f32),
            pltpu.VMEM((chunk,), jnp.int32),
            pltpu.VMEM((chunk,), f32),
            pltpu.VMEM((chunk,), jnp.int32),
            pltpu.VMEM((chunk,), f32),
            pltpu.SemaphoreType.DMA,
            pltpu.SemaphoreType.DMA,
        ],
    )
    outp = sc_agg(htp, ex, pk)

    out = pl.pallas_call(
        _fin_body,
        out_shape=jax.ShapeDtypeStruct((N, Dh), f32),
    )(outp.reshape(Dh, N), rden)
    return out
